# baseline (speedup 1.0000x reference)
import functools

import jax
import jax.numpy as jnp
from jax import lax
from jax.experimental import pallas as pl
from jax.experimental.pallas import tpu as pltpu
from jax.experimental.pallas import tpu_sc as plsc

VOCAB = 1000000
EMBED = 64
BATCH = 16384
C = 6
PW = 2 * EMBED

_info = plsc.get_sparse_core_info()
NC, NS, L = _info.num_cores, _info.num_subcores, _info.num_lanes
NW = NC * NS
B_PER_W = BATCH // NW
CH = 128
NCHUNK = B_PER_W // CH

DP_ROWS = 4000


def _depad_body(lo_ref, hi_ref, out_ref):
    out_ref[...] = jnp.concatenate([lo_ref[...], hi_ref[...]], axis=1)


def _depad(table):
    half_blocks = (VOCAB // 2) // DP_ROWS
    return pl.pallas_call(
        _depad_body,
        grid=(half_blocks,),
        in_specs=[
            pl.BlockSpec((DP_ROWS, EMBED), lambda g: (g, 0)),
            pl.BlockSpec((DP_ROWS, EMBED), lambda g: (g + half_blocks, 0)),
        ],
        out_specs=pl.BlockSpec((DP_ROWS, PW), lambda g: (g, 0)),
        out_shape=jax.ShapeDtypeStruct((VOCAB // 2, PW), jnp.float32),
    )(table, table)


def _sc_kernel(tgt_idx_hbm, tgt_off_hbm, ctx_idx_hbm, ctx_off_hbm,
               tgt_tab_hbm, ctx_tab_hbm, out_hbm,
               tidx_v, toff_v, cidx_v, coff_v, trows_v, crows_v, out_v,
               ptile, sem):
    wid = lax.axis_index("s") * NC + lax.axis_index("c")
    lanes = lax.iota(jnp.int32, L)

    for ch in range(NCHUNK):
        tbase = wid * B_PER_W + ch * CH
        pltpu.sync_copy(tgt_idx_hbm.at[pl.ds(tbase, CH)], tidx_v)
        pltpu.sync_copy(tgt_off_hbm.at[pl.ds(tbase, CH)], toff_v)
        pltpu.sync_copy(ctx_idx_hbm.at[pl.ds(tbase * C, CH * C)], cidx_v)
        pltpu.sync_copy(ctx_off_hbm.at[pl.ds(tbase * C, CH * C)], coff_v)

        cp_t = pltpu.make_async_copy(tgt_tab_hbm.at[tidx_v], trows_v, sem)
        cp_t.start()
        cps = []
        for j in range(C):
            cp = pltpu.make_async_copy(
                ctx_tab_hbm.at[cidx_v.at[pl.ds(j * CH, CH)]],
                crows_v.at[pl.ds(j * CH, CH)], sem)
            cp.start()
            cps.append(cp)
        cp_t.wait()
        for cp in cps:
            cp.wait()

        IB = 8
        NROW = IB * C
        col0 = lanes * L

        def block_body(b, _):
            i0 = b * IB
            for ii in range(IB):
                i = i0 + ii
                ivec = jnp.full((L,), i, jnp.int32)
                tof = plsc.load_gather(toff_v, [ivec])
                t = [
                    plsc.load_gather(trows_v, [ivec, tof + (k * L) + lanes])
                    for k in range(EMBED // L)
                ]
                for c in range(C):
                    row = i * C + c
                    rvec = jnp.full((L,), row, jnp.int32)
                    cof = plsc.load_gather(coff_v, [rvec])
                    p = plsc.load_gather(crows_v, [rvec, cof + lanes]) * t[0]
                    for k in range(1, EMBED // L):
                        p = p + plsc.load_gather(
                            crows_v, [rvec, cof + (k * L) + lanes]) * t[k]
                    ptile[pl.ds((ii * C + c) * L, L)] = p
            for g in range(NROW // L):
                acc = plsc.load_gather(ptile, [col0 + (g * L * L)])
                for j in range(1, L):
                    acc = acc + plsc.load_gather(
                        ptile, [col0 + (g * L * L + j)])
                out_v[pl.ds(i0 * C + g * L, L)] = acc
            return 0

        lax.fori_loop(0, CH // IB, block_body, 0)

        out_base = (wid * NCHUNK + ch) * CH * C
        pltpu.sync_copy(out_v, out_hbm.at[pl.ds(out_base, CH * C)])


def kernel(target, context, target_table, context_table):
    tgt_idx = target.reshape(BATCH).astype(jnp.int32)
    ctx_idx = context.reshape(BATCH * C).astype(jnp.int32)
    half = VOCAB // 2
    tgt_pair = tgt_idx % half
    tgt_off = (tgt_idx // half) * EMBED
    ctx_pair = ctx_idx % half
    ctx_off = (ctx_idx // half) * EMBED
    tgt_tab = _depad(target_table)
    ctx_tab = _depad(context_table)

    mesh = plsc.VectorSubcoreMesh(core_axis_name="c", subcore_axis_name="s")
    run = functools.partial(
        pl.kernel,
        mesh=mesh,
        compiler_params=pltpu.CompilerParams(needs_layout_passes=False),
        out_type=jax.ShapeDtypeStruct((BATCH * C,), jnp.float32),
        scratch_types=[
            pltpu.VMEM((CH,), jnp.int32),
            pltpu.VMEM((CH,), jnp.int32),
            pltpu.VMEM((CH * C,), jnp.int32),
            pltpu.VMEM((CH * C,), jnp.int32),
            pltpu.VMEM((CH, PW), jnp.float32),
            pltpu.VMEM((CH * C, PW), jnp.float32),
            pltpu.VMEM((CH * C,), jnp.float32),
            pltpu.VMEM((8 * C * L,), jnp.float32),
            pltpu.SemaphoreType.DMA,
        ],
    )(_sc_kernel)
    out = run(tgt_pair, tgt_off, ctx_pair, ctx_off, tgt_tab, ctx_tab)
    return out.reshape(BATCH, C)


# speedup vs baseline: 1.0147x; 1.0147x over previous
import functools

import jax
import jax.numpy as jnp
from jax import lax
from jax.experimental import pallas as pl
from jax.experimental.pallas import tpu as pltpu
from jax.experimental.pallas import tpu_sc as plsc

VOCAB = 1000000
EMBED = 64
BATCH = 16384
C = 6
PW = 2 * EMBED

_info = plsc.get_sparse_core_info()
NC, NS, L = _info.num_cores, _info.num_subcores, _info.num_lanes
NW = NC * NS
B_PER_W = BATCH // NW
CH = 128
NCHUNK = B_PER_W // CH

DP_ROWS = 20000


def _depad_body(lo_ref, hi_ref, out_ref):
    out_ref[...] = jnp.concatenate([lo_ref[...], hi_ref[...]], axis=1)


def _depad(table):
    half_blocks = (VOCAB // 2) // DP_ROWS
    return pl.pallas_call(
        _depad_body,
        grid=(half_blocks,),
        in_specs=[
            pl.BlockSpec((DP_ROWS, EMBED), lambda g: (g, 0)),
            pl.BlockSpec((DP_ROWS, EMBED), lambda g: (g + half_blocks, 0)),
        ],
        out_specs=pl.BlockSpec((DP_ROWS, PW), lambda g: (g, 0)),
        out_shape=jax.ShapeDtypeStruct((VOCAB // 2, PW), jnp.float32),
    )(table, table)


def _sc_kernel(tgt_idx_hbm, tgt_off_hbm, ctx_idx_hbm, ctx_off_hbm,
               tgt_tab_hbm, ctx_tab_hbm, out_hbm,
               tidx_v, toff_v, cidx_v, coff_v, trows_v, crows_v, out_v,
               ptile, sem):
    wid = lax.axis_index("s") * NC + lax.axis_index("c")
    lanes = lax.iota(jnp.int32, L)

    for ch in range(NCHUNK):
        tbase = wid * B_PER_W + ch * CH
        pltpu.sync_copy(tgt_idx_hbm.at[pl.ds(tbase, CH)], tidx_v)
        pltpu.sync_copy(tgt_off_hbm.at[pl.ds(tbase, CH)], toff_v)
        pltpu.sync_copy(ctx_idx_hbm.at[pl.ds(tbase * C, CH * C)], cidx_v)
        pltpu.sync_copy(ctx_off_hbm.at[pl.ds(tbase * C, CH * C)], coff_v)

        cp_t = pltpu.make_async_copy(tgt_tab_hbm.at[tidx_v], trows_v, sem)
        cp_t.start()
        cps = []
        for j in range(C):
            cp = pltpu.make_async_copy(
                ctx_tab_hbm.at[cidx_v.at[pl.ds(j * CH, CH)]],
                crows_v.at[pl.ds(j * CH, CH)], sem)
            cp.start()
            cps.append(cp)
        cp_t.wait()
        for cp in cps:
            cp.wait()

        IB = 8
        NROW = IB * C
        col0 = lanes * L

        def block_body(b, _):
            i0 = b * IB
            for ii in range(IB):
                i = i0 + ii
                ivec = jnp.full((L,), i, jnp.int32)
                tof = plsc.load_gather(toff_v, [ivec])
                t = [
                    plsc.load_gather(trows_v, [ivec, tof + (k * L) + lanes])
                    for k in range(EMBED // L)
                ]
                for c in range(C):
                    row = i * C + c
                    rvec = jnp.full((L,), row, jnp.int32)
                    cof = plsc.load_gather(coff_v, [rvec])
                    p = plsc.load_gather(crows_v, [rvec, cof + lanes]) * t[0]
                    for k in range(1, EMBED // L):
                        p = p + plsc.load_gather(
                            crows_v, [rvec, cof + (k * L) + lanes]) * t[k]
                    ptile[pl.ds((ii * C + c) * L, L)] = p
            for g in range(NROW // L):
                acc = plsc.load_gather(ptile, [col0 + (g * L * L)])
                for j in range(1, L):
                    acc = acc + plsc.load_gather(
                        ptile, [col0 + (g * L * L + j)])
                out_v[pl.ds(i0 * C + g * L, L)] = acc
            return 0

        lax.fori_loop(0, CH // IB, block_body, 0)

        out_base = (wid * NCHUNK + ch) * CH * C
        pltpu.sync_copy(out_v, out_hbm.at[pl.ds(out_base, CH * C)])


def kernel(target, context, target_table, context_table):
    tgt_idx = target.reshape(BATCH).astype(jnp.int32)
    ctx_idx = context.reshape(BATCH * C).astype(jnp.int32)
    half = VOCAB // 2
    tgt_pair = tgt_idx % half
    tgt_off = (tgt_idx // half) * EMBED
    ctx_pair = ctx_idx % half
    ctx_off = (ctx_idx // half) * EMBED
    tgt_tab = _depad(target_table)
    ctx_tab = _depad(context_table)

    mesh = plsc.VectorSubcoreMesh(core_axis_name="c", subcore_axis_name="s")
    run = functools.partial(
        pl.kernel,
        mesh=mesh,
        compiler_params=pltpu.CompilerParams(needs_layout_passes=False),
        out_type=jax.ShapeDtypeStruct((BATCH * C,), jnp.float32),
        scratch_types=[
            pltpu.VMEM((CH,), jnp.int32),
            pltpu.VMEM((CH,), jnp.int32),
            pltpu.VMEM((CH * C,), jnp.int32),
            pltpu.VMEM((CH * C,), jnp.int32),
            pltpu.VMEM((CH, PW), jnp.float32),
            pltpu.VMEM((CH * C, PW), jnp.float32),
            pltpu.VMEM((CH * C,), jnp.float32),
            pltpu.VMEM((8 * C * L,), jnp.float32),
            pltpu.SemaphoreType.DMA,
        ],
    )(_sc_kernel)
    out = run(tgt_pair, tgt_off, ctx_pair, ctx_off, tgt_tab, ctx_tab)
    return out.reshape(BATCH, C)


# speedup vs baseline: 1.1469x; 1.1304x over previous
import functools

import jax
import jax.numpy as jnp
from jax import lax
from jax.experimental import pallas as pl
from jax.experimental.pallas import tpu as pltpu
from jax.experimental.pallas import tpu_sc as plsc

VOCAB = 1000000
EMBED = 64
BATCH = 16384
C = 6

_info = plsc.get_sparse_core_info()
NC, NS, L = _info.num_cores, _info.num_subcores, _info.num_lanes
NW = NC * NS
B_PER_W = BATCH // NW
CH = 128
NCHUNK = B_PER_W // CH


def _sc_kernel(tgt_idx_hbm, ctx_idx_hbm, tgt_tab_hbm, ctx_tab_hbm, out_hbm,
               tidx_v, cidx_v, trows_v, crows_v, out_v, ptile, sem):
    wid = lax.axis_index("s") * NC + lax.axis_index("c")

    for ch in range(NCHUNK):
        tbase = wid * B_PER_W + ch * CH
        pltpu.sync_copy(tgt_idx_hbm.at[pl.ds(tbase, CH)], tidx_v)
        pltpu.sync_copy(ctx_idx_hbm.at[pl.ds(tbase * C, CH * C)], cidx_v)

        cp_t = pltpu.make_async_copy(tgt_tab_hbm.at[tidx_v], trows_v, sem)
        cp_t.start()
        cps = []
        for j in range(C):
            cp = pltpu.make_async_copy(
                ctx_tab_hbm.at[cidx_v.at[pl.ds(j * CH, CH)]],
                crows_v.at[pl.ds(j * CH, CH)], sem)
            cp.start()
            cps.append(cp)
        cp_t.wait()
        for cp in cps:
            cp.wait()

        IB = 8
        NROW = IB * C
        col0 = lax.iota(jnp.int32, L) * L

        def block_body(b, _):
            i0 = b * IB
            for ii in range(IB):
                i = i0 + ii
                t = [trows_v[i, pl.ds(k * L, L)] for k in range(EMBED // L)]
                for c in range(C):
                    row = i * C + c
                    p = crows_v[row, pl.ds(0, L)] * t[0]
                    for k in range(1, EMBED // L):
                        p = p + crows_v[row, pl.ds(k * L, L)] * t[k]
                    ptile[pl.ds((ii * C + c) * L, L)] = p
            for g in range(NROW // L):
                acc = plsc.load_gather(ptile, [col0 + (g * L * L)])
                for j in range(1, L):
                    acc = acc + plsc.load_gather(
                        ptile, [col0 + (g * L * L + j)])
                out_v[pl.ds(i0 * C + g * L, L)] = acc
            return 0

        lax.fori_loop(0, CH // IB, block_body, 0)

        out_base = (wid * NCHUNK + ch) * CH * C
        pltpu.sync_copy(out_v, out_hbm.at[pl.ds(out_base, CH * C)])


def kernel(target, context, target_table, context_table):
    tgt_idx = target.reshape(BATCH).astype(jnp.int32)
    ctx_idx = context.reshape(BATCH * C).astype(jnp.int32)

    mesh = plsc.VectorSubcoreMesh(core_axis_name="c", subcore_axis_name="s")
    run = functools.partial(
        pl.kernel,
        mesh=mesh,
        compiler_params=pltpu.CompilerParams(
            needs_layout_passes=False, use_tc_tiling_on_sc=False),
        out_type=jax.ShapeDtypeStruct((BATCH * C,), jnp.float32),
        scratch_types=[
            pltpu.VMEM((CH,), jnp.int32),
            pltpu.VMEM((CH * C,), jnp.int32),
            pltpu.VMEM((CH, EMBED), jnp.float32),
            pltpu.VMEM((CH * C, EMBED), jnp.float32),
            pltpu.VMEM((CH * C,), jnp.float32),
            pltpu.VMEM((8 * C * L,), jnp.float32),
            pltpu.SemaphoreType.DMA,
        ],
    )(_sc_kernel)
    out = run(tgt_idx, ctx_idx, target_table, context_table)
    return out.reshape(BATCH, C)
